# baseline (speedup 1.0000x reference)
import functools

import jax
import jax.numpy as jnp
import numpy as np
from jax import lax
from jax.experimental import pallas as pl
from jax.experimental.pallas import tpu as pltpu
from jax.experimental.pallas import tpu_sc as plsc

_VOCAB = 100000
_D = 128
_B = 4096
_H = 50
_ROWS = _B * _H
_KEEP_P = 0.7

_NC, _NS, _L = 2, 16, 16
_NW = _NC * _NS
_RPW = _ROWS // _NW
_CH = 256
_NCHUNK = _RPW // _CH
_WPC = _CH * 8


def _make_maskwords() -> np.ndarray:
    def _draw():
        return np.asarray(
            jax.random.bernoulli(jax.random.key(42), _KEEP_P, (_B, _H, _D)))
    try:
        with jax.default_device(jax.devices("cpu")[0]):
            keep = _draw()
    except Exception:
        keep = _draw()
    bits = keep.transpose(1, 0, 2).reshape(_ROWS * 8, 16).astype(np.uint32)
    return (bits << np.arange(16, dtype=np.uint32)).sum(-1).astype(np.int32)


_MASKW = _make_maskwords()

_mesh = plsc.VectorSubcoreMesh(core_axis_name="c", subcore_axis_name="s")


@functools.partial(
    pl.kernel,
    out_type=jax.ShapeDtypeStruct((_ROWS, _D), jnp.float32),
    mesh=_mesh,
    scratch_types=[
        pltpu.VMEM((_RPW // 128, 128), jnp.int32),
        pltpu.VMEM((_CH, _D), jnp.float32),
        pltpu.VMEM((_CH, _D), jnp.float32),
        pltpu.VMEM((_WPC,), jnp.int32),
        pltpu.VMEM((_WPC,), jnp.int32),
        pltpu.SemaphoreType.DMA,
        pltpu.SemaphoreType.DMA,
        pltpu.SemaphoreType.DMA,
        pltpu.SemaphoreType.DMA,
    ],
)
def _emb_kernel(table_hbm, idx_hbm, maskw_hbm, out_hbm, idx_v,
                rows0, rows1, words0, words1, sg0, sg1, sw0, sw1):
    wid = lax.axis_index("s") * _NC + lax.axis_index("c")
    row0w = wid * _RPW
    lane = lax.iota(jnp.int32, 16)
    scale = jnp.float32(1.0 / _KEEP_P)
    pltpu.sync_copy(idx_hbm.at[wid], idx_v)

    bufs = ((rows0, words0, sg0, sw0), (rows1, words1, sg1, sw1))
    _GS = _CH // 128

    def issue_chunk(c, rows_b, words_b, sg):
        row0 = row0w + c * _CH
        pltpu.async_copy(maskw_hbm.at[pl.ds(row0 * 8, _WPC)], words_b, sg)
        for j in range(_GS):
            pltpu.async_copy(
                table_hbm.at[idx_v.at[c * _GS + j]],
                rows_b.at[pl.ds(j * 128, 128)], sg)

    def wait_chunk(rows_b, words_b, sg):
        pltpu.make_async_copy(maskw_hbm.at[pl.ds(0, _WPC)], words_b, sg).wait()
        for j in range(_GS):
            pltpu.make_async_copy(
                table_hbm.at[idx_v.at[j]],
                rows_b.at[pl.ds(j * 128, 128)], sg).wait()

    lanebit = jnp.left_shift(jnp.int32(1), lane)
    zerov = jnp.zeros((16,), jnp.float32)

    def vloop(rows_b, words_b):
        @plsc.parallel_loop(0, _CH // 2, unroll=2)
        def pair_body(p):
            wvec = words_b[pl.ds(p * 16, 16)]
            r0 = 2 * p
            for v in range(16):
                word = wvec[v]
                row = r0 + (v // 8)
                sl = pl.ds((v % 8) * 16, 16)
                keep = (word & lanebit) != 0
                scaled = rows_b[row, sl] * scale
                rows_b[row, sl] = jnp.where(keep, scaled, zerov)

    issue_chunk(0, rows0, words0, sg0)

    def gbody(g, carry):
        for par in range(2):
            c = 2 * g + par
            rows_p, words_p, sg_p, sw_p = bufs[par]
            rows_o, words_o, sg_o, sw_o = bufs[1 - par]
            wait_chunk(rows_p, words_p, sg_p)

            @pl.when(c >= 1)
            def _():
                pltpu.make_async_copy(
                    rows_o, out_hbm.at[pl.ds(row0w, _CH)], sw_o).wait()

            @pl.when(c + 1 < _NCHUNK)
            def _():
                issue_chunk(c + 1, rows_o, words_o, sg_o)

            vloop(rows_p, words_p)
            pltpu.async_copy(
                rows_p, out_hbm.at[pl.ds(row0w + c * _CH, _CH)], sw_p)
        return carry

    lax.fori_loop(0, _NCHUNK // 2, gbody, 0)
    c = _NCHUNK - 1
    rows_p, words_p, sg_p, sw_p = bufs[c % 2]
    rows_o, words_o, sg_o, sw_o = bufs[1 - c % 2]
    wait_chunk(rows_p, words_p, sg_p)
    pltpu.make_async_copy(rows_o, out_hbm.at[pl.ds(row0w, _CH)], sw_o).wait()
    vloop(rows_p, words_p)
    pltpu.async_copy(rows_p, out_hbm.at[pl.ds(row0w + c * _CH, _CH)], sw_p)
    pltpu.make_async_copy(rows_p, out_hbm.at[pl.ds(row0w, _CH)], sw_p).wait()


def kernel(w_tensor, table):
    w_t = jnp.transpose(w_tensor)
    idx3 = w_t.reshape(_NW, _RPW // 128, 128)
    out = _emb_kernel(table, idx3, jnp.asarray(_MASKW))
    return jnp.transpose(out.reshape(_H, _B, _D), (1, 0, 2))


# speedup vs baseline: 1.1859x; 1.1859x over previous
import functools

import jax
import jax.numpy as jnp
import numpy as np
from jax import lax
from jax.experimental import pallas as pl
from jax.experimental.pallas import tpu as pltpu
from jax.experimental.pallas import tpu_sc as plsc

_VOCAB = 100000
_D = 128
_B = 4096
_H = 50
_ROWS = _B * _H
_KEEP_P = 0.7

_NC, _NS, _L = 2, 16, 16
_NW = _NC * _NS
_RPW = _ROWS // _NW
_CH = 256
_NCHUNK = _RPW // _CH
_WPC = _CH * 8


def _threefry2x32(k0, k1, x0, x1):
    rot = ((13, 15, 26, 6), (17, 29, 16, 24))
    ks = (np.uint32(k0), np.uint32(k1),
          np.uint32(k0) ^ np.uint32(k1) ^ np.uint32(0x1BD11BDA))
    x0 = (x0 + ks[0]).astype(np.uint32)
    x1 = (x1 + ks[1]).astype(np.uint32)
    for d in range(5):
        for r in rot[d % 2]:
            x0 = (x0 + x1).astype(np.uint32)
            x1 = ((x1 << np.uint32(r)) | (x1 >> np.uint32(32 - r))).astype(np.uint32)
            x1 = x1 ^ x0
        x0 = (x0 + ks[(d + 1) % 3]).astype(np.uint32)
        x1 = (x1 + ks[(d + 2) % 3] + np.uint32(d + 1)).astype(np.uint32)
    return x0, x1


def _make_maskwords() -> np.ndarray:
    n = _B * _H * _D
    x0, x1 = _threefry2x32(0, 42, np.zeros(n, np.uint32),
                           np.arange(n, dtype=np.uint32))
    fbits = ((x0 ^ x1) >> np.uint32(9)) | np.uint32(0x3F800000)
    keep = (fbits.view(np.float32) - np.float32(1.0)) < np.float32(_KEEP_P)
    bits = keep.reshape(_B, _H, _D).transpose(1, 0, 2)
    bits = bits.reshape(_ROWS * 8, 16).astype(np.uint32)
    return (bits << np.arange(16, dtype=np.uint32)).sum(-1).astype(np.int32)


_MASKW = _make_maskwords()

_mesh = plsc.VectorSubcoreMesh(core_axis_name="c", subcore_axis_name="s")


@functools.partial(
    pl.kernel,
    out_type=jax.ShapeDtypeStruct((_ROWS, _D), jnp.float32),
    mesh=_mesh,
    scratch_types=[
        pltpu.VMEM((_RPW // 128, 128), jnp.int32),
        pltpu.VMEM((_CH, _D), jnp.float32),
        pltpu.VMEM((_CH, _D), jnp.float32),
        pltpu.VMEM((_WPC,), jnp.int32),
        pltpu.VMEM((_WPC,), jnp.int32),
        pltpu.SemaphoreType.DMA,
        pltpu.SemaphoreType.DMA,
        pltpu.SemaphoreType.DMA,
        pltpu.SemaphoreType.DMA,
    ],
)
def _emb_kernel(table_hbm, idx_hbm, maskw_hbm, out_hbm, idx_v,
                rows0, rows1, words0, words1, sg0, sg1, sw0, sw1):
    wid = lax.axis_index("s") * _NC + lax.axis_index("c")
    row0w = wid * _RPW
    lane = lax.iota(jnp.int32, 16)
    scale = jnp.float32(1.0 / _KEEP_P)
    pltpu.sync_copy(idx_hbm.at[wid], idx_v)

    bufs = ((rows0, words0, sg0, sw0), (rows1, words1, sg1, sw1))
    _GS = _CH // 128

    def issue_chunk(c, rows_b, words_b, sg):
        row0 = row0w + c * _CH
        pltpu.async_copy(maskw_hbm.at[pl.ds(row0 * 8, _WPC)], words_b, sg)
        for j in range(_GS):
            pltpu.async_copy(
                table_hbm.at[idx_v.at[c * _GS + j]],
                rows_b.at[pl.ds(j * 128, 128)], sg)

    def wait_chunk(rows_b, words_b, sg):
        pltpu.make_async_copy(maskw_hbm.at[pl.ds(0, _WPC)], words_b, sg).wait()
        for j in range(_GS):
            pltpu.make_async_copy(
                table_hbm.at[idx_v.at[j]],
                rows_b.at[pl.ds(j * 128, 128)], sg).wait()

    def vloop(rows_b, words_b):
        @plsc.parallel_loop(0, _CH // 2, unroll=2)
        def pair_body(p):
            wvec = words_b[pl.ds(p * 16, 16)]
            r0 = 2 * p
            for v in range(16):
                word = wvec[v]
                row = r0 + (v // 8)
                sl = pl.ds((v % 8) * 16, 16)
                bits = (word >> lane) & 1
                mul = bits.astype(jnp.float32) * scale
                rows_b[row, sl] = rows_b[row, sl] * mul

    issue_chunk(0, rows0, words0, sg0)

    def gbody(g, carry):
        for par in range(2):
            c = 2 * g + par
            rows_p, words_p, sg_p, sw_p = bufs[par]
            rows_o, words_o, sg_o, sw_o = bufs[1 - par]
            wait_chunk(rows_p, words_p, sg_p)

            @pl.when(c >= 1)
            def _():
                pltpu.make_async_copy(
                    rows_o, out_hbm.at[pl.ds(row0w, _CH)], sw_o).wait()

            @pl.when(c + 1 < _NCHUNK)
            def _():
                issue_chunk(c + 1, rows_o, words_o, sg_o)

            vloop(rows_p, words_p)
            pltpu.async_copy(
                rows_p, out_hbm.at[pl.ds(row0w + c * _CH, _CH)], sw_p)
        return carry

    lax.fori_loop(0, _NCHUNK // 2, gbody, 0)
    c = _NCHUNK - 1
    rows_p, words_p, sg_p, sw_p = bufs[c % 2]
    rows_o, words_o, sg_o, sw_o = bufs[1 - c % 2]
    wait_chunk(rows_p, words_p, sg_p)
    pltpu.make_async_copy(rows_o, out_hbm.at[pl.ds(row0w, _CH)], sw_o).wait()
    vloop(rows_p, words_p)
    pltpu.async_copy(rows_p, out_hbm.at[pl.ds(row0w + c * _CH, _CH)], sw_p)
    pltpu.make_async_copy(rows_p, out_hbm.at[pl.ds(row0w, _CH)], sw_p).wait()


def kernel(w_tensor, table):
    w_t = jnp.transpose(w_tensor)
    idx3 = w_t.reshape(_NW, _RPW // 128, 128)
    out = _emb_kernel(table, idx3, jnp.asarray(_MASKW))
    return jnp.transpose(out.reshape(_H, _B, _D), (1, 0, 2))


# speedup vs baseline: 1.3259x; 1.1180x over previous
import functools

import jax
import jax.numpy as jnp
import numpy as np
from jax import lax
from jax.experimental import pallas as pl
from jax.experimental.pallas import tpu as pltpu
from jax.experimental.pallas import tpu_sc as plsc

_VOCAB = 100000
_D = 128
_B = 4096
_H = 50
_ROWS = _B * _H
_KEEP_P = 0.7

_NC, _NS, _L = 2, 16, 16
_NW = _NC * _NS
_RPW = _ROWS // _NW
_CH = 256
_NCHUNK = _RPW // _CH
_WPC = _CH * 8


def _threefry2x32(k0, k1, x0, x1):
    rot = ((13, 15, 26, 6), (17, 29, 16, 24))
    ks = (np.uint32(k0), np.uint32(k1),
          np.uint32(k0) ^ np.uint32(k1) ^ np.uint32(0x1BD11BDA))
    x0 = (x0 + ks[0]).astype(np.uint32)
    x1 = (x1 + ks[1]).astype(np.uint32)
    for d in range(5):
        for r in rot[d % 2]:
            x0 = (x0 + x1).astype(np.uint32)
            x1 = ((x1 << np.uint32(r)) | (x1 >> np.uint32(32 - r))).astype(np.uint32)
            x1 = x1 ^ x0
        x0 = (x0 + ks[(d + 1) % 3]).astype(np.uint32)
        x1 = (x1 + ks[(d + 2) % 3] + np.uint32(d + 1)).astype(np.uint32)
    return x0, x1


def _make_maskwords() -> np.ndarray:
    n = _B * _H * _D
    x0, x1 = _threefry2x32(0, 42, np.zeros(n, np.uint32),
                           np.arange(n, dtype=np.uint32))
    fbits = ((x0 ^ x1) >> np.uint32(9)) | np.uint32(0x3F800000)
    keep = (fbits.view(np.float32) - np.float32(1.0)) < np.float32(_KEEP_P)
    bits = keep.reshape(_B, _H, _D).transpose(1, 0, 2)
    bits = bits.reshape(_ROWS * 8, 16).astype(np.uint32)
    return (bits << np.arange(16, dtype=np.uint32)).sum(-1).astype(np.int32)


_MASKW = _make_maskwords()

_mesh = plsc.VectorSubcoreMesh(core_axis_name="c", subcore_axis_name="s")


@functools.partial(
    pl.kernel,
    out_type=jax.ShapeDtypeStruct((_ROWS, _D), jnp.float32),
    mesh=_mesh,
    scratch_types=[
        pltpu.VMEM((_RPW // 128, 128), jnp.int32),
        pltpu.VMEM((_CH, _D), jnp.float32),
        pltpu.VMEM((_CH, _D), jnp.float32),
        pltpu.VMEM((_WPC,), jnp.int32),
        pltpu.VMEM((_WPC,), jnp.int32),
        pltpu.SemaphoreType.DMA,
        pltpu.SemaphoreType.DMA,
        pltpu.SemaphoreType.DMA,
        pltpu.SemaphoreType.DMA,
    ],
)
def _emb_kernel(table_hbm, idx_hbm, maskw_hbm, out_hbm, idx_v,
                rows0, rows1, words0, words1, sg0, sg1, sw0, sw1):
    wid = lax.axis_index("s") * _NC + lax.axis_index("c")
    row0w = wid * _RPW
    lane = lax.iota(jnp.int32, 16)
    scale = jnp.float32(1.0 / _KEEP_P)
    revlane = jnp.int32(31) - lane
    lane31 = jnp.full((16,), 31, jnp.int32)
    scalebits = jnp.full(
        (16,), int(np.float32(1.0 / _KEEP_P).view(np.int32)), jnp.int32)
    pltpu.sync_copy(idx_hbm.at[wid], idx_v)

    bufs = ((rows0, words0, sg0, sw0), (rows1, words1, sg1, sw1))
    _GS = _CH // 128

    def issue_chunk(c, rows_b, words_b, sg):
        row0 = row0w + c * _CH
        pltpu.async_copy(maskw_hbm.at[pl.ds(row0 * 8, _WPC)], words_b, sg)
        for j in range(_GS):
            pltpu.async_copy(
                table_hbm.at[idx_v.at[c * _GS + j]],
                rows_b.at[pl.ds(j * 128, 128)], sg)

    def wait_chunk(rows_b, words_b, sg):
        pltpu.make_async_copy(maskw_hbm.at[pl.ds(0, _WPC)], words_b, sg).wait()
        for j in range(_GS):
            pltpu.make_async_copy(
                table_hbm.at[idx_v.at[j]],
                rows_b.at[pl.ds(j * 128, 128)], sg).wait()

    def vloop(rows_b, words_b):
        @plsc.parallel_loop(0, _CH // 2, unroll=2)
        def pair_body(p):
            wvec = words_b[pl.ds(p * 16, 16)]
            r0 = 2 * p
            for v in range(16):
                word = wvec[v]
                row = r0 + (v // 8)
                sl = pl.ds((v % 8) * 16, 16)
                keepm = (word << revlane) >> lane31
                mulf = lax.bitcast_convert_type(keepm & scalebits, jnp.float32)
                rows_b[row, sl] = rows_b[row, sl] * mulf

    issue_chunk(0, rows0, words0, sg0)

    def gbody(g, carry):
        for par in range(2):
            c = 2 * g + par
            rows_p, words_p, sg_p, sw_p = bufs[par]
            rows_o, words_o, sg_o, sw_o = bufs[1 - par]
            wait_chunk(rows_p, words_p, sg_p)

            @pl.when(c >= 1)
            def _():
                pltpu.make_async_copy(
                    rows_o, out_hbm.at[pl.ds(row0w, _CH)], sw_o).wait()

            @pl.when(c + 1 < _NCHUNK)
            def _():
                issue_chunk(c + 1, rows_o, words_o, sg_o)

            vloop(rows_p, words_p)
            pltpu.async_copy(
                rows_p, out_hbm.at[pl.ds(row0w + c * _CH, _CH)], sw_p)
        return carry

    lax.fori_loop(0, _NCHUNK // 2, gbody, 0)
    c = _NCHUNK - 1
    rows_p, words_p, sg_p, sw_p = bufs[c % 2]
    rows_o, words_o, sg_o, sw_o = bufs[1 - c % 2]
    wait_chunk(rows_p, words_p, sg_p)
    pltpu.make_async_copy(rows_o, out_hbm.at[pl.ds(row0w, _CH)], sw_o).wait()
    vloop(rows_p, words_p)
    pltpu.async_copy(rows_p, out_hbm.at[pl.ds(row0w + c * _CH, _CH)], sw_p)
    pltpu.make_async_copy(rows_p, out_hbm.at[pl.ds(row0w, _CH)], sw_p).wait()


def kernel(w_tensor, table):
    w_t = jnp.transpose(w_tensor)
    idx3 = w_t.reshape(_NW, _RPW // 128, 128)
    out = _emb_kernel(table, idx3, jnp.asarray(_MASKW))
    return jnp.transpose(out.reshape(_H, _B, _D), (1, 0, 2))
